# speedup vs baseline: 1.7040x; 1.7040x over previous
import jax
import jax.numpy as jnp
from jax.experimental import pallas as pl
from jax.experimental.pallas import tpu as pltpu

B = 16
V = 1_000_000
C = 131072
K = (V + C - 1) // C
NEG = -1e30


def _body(x_ref, a_ref, o_ref, m_acc, s_acc, g_acc):
    k = pl.program_id(0)

    @pl.when(k == 0)
    def _init():
        m_acc[...] = jnp.full((B, 128), NEG, jnp.float32)
        s_acc[...] = jnp.zeros((B, 128), jnp.float32)
        g_acc[...] = jnp.zeros((B, 128), jnp.float32)

    x = x_ref[...].reshape(B, C // 128, 128)
    col = (
        jax.lax.broadcasted_iota(jnp.int32, (B, C // 128, 128), 1) * 128
        + jax.lax.broadcasted_iota(jnp.int32, (B, C // 128, 128), 2)
        + k * C
    )
    valid = col < V
    xm = jnp.where(valid, x, NEG)

    bm = jnp.max(xm, axis=1)
    m_old = m_acc[...]
    m_new = jnp.maximum(m_old, bm)
    e = jnp.exp(xm - m_new[:, None, :])
    bs = jnp.sum(e, axis=1)
    s_acc[...] = s_acc[...] * jnp.exp(m_old - m_new) + bs
    m_acc[...] = m_new

    a = a_ref[...].reshape(B, 1, 1)
    g_acc[...] += jnp.sum(jnp.where(col == a, xm, 0.0), axis=1)

    @pl.when(k == K - 1)
    def _fin():
        m = m_acc[...]
        gmax = jnp.max(m, axis=1, keepdims=True)
        st = jnp.sum(s_acc[...] * jnp.exp(m - gmax), axis=1, keepdims=True)
        gv = jnp.sum(g_acc[...], axis=1, keepdims=True)
        o_ref[...] = gv - (gmax + jnp.log(st))


def kernel(logits, actions):
    a = actions.astype(jnp.int32)
    out = pl.pallas_call(
        _body,
        grid=(K,),
        in_specs=[
            pl.BlockSpec((B, C), lambda k: (0, k)),
            pl.BlockSpec((B, 1), lambda k: (0, 0)),
        ],
        out_specs=pl.BlockSpec((B, 1), lambda k: (0, 0)),
        out_shape=jax.ShapeDtypeStruct((B, 1), jnp.float32),
        scratch_shapes=[
            pltpu.VMEM((B, 128), jnp.float32),
            pltpu.VMEM((B, 128), jnp.float32),
            pltpu.VMEM((B, 128), jnp.float32),
        ],
    )(logits, a)
    return out
